# baseline (speedup 1.0000x reference)
import functools

import jax
import jax.numpy as jnp
from jax import lax
from jax.experimental import pallas as pl
from jax.experimental.pallas import tpu as pltpu
from jax.experimental.pallas import tpu_sc as plsc

_VOCAB = 100000
_D = 768
_B = 4
_S = 4096
_N = _B * _S
_NW = 32
_PER_W = _N // _NW
_CHUNK = 64
_NCHUNK = _PER_W // _CHUNK
_LANES = 16
_VECS = _D // _LANES


def _build_sc_kernel():
  mesh = plsc.VectorSubcoreMesh(core_axis_name="c", subcore_axis_name="s")

  @functools.partial(
      pl.kernel,
      mesh=mesh,
      out_type=jax.ShapeDtypeStruct((_N, _D), jnp.float32),
      scratch_types=[
          pltpu.VMEM((_NCHUNK, _CHUNK), jnp.int32),
          pltpu.VMEM((_CHUNK, _D), jnp.float32),
          pltpu.VMEM((_CHUNK, _D), jnp.float32),
          pltpu.SemaphoreType.DMA,
          pltpu.SemaphoreType.DMA,
      ],
  )
  def embed(x_hbm, tok_hbm, pos_hbm, out_hbm, idx_v, tok_v, acc_v, gsem, psem):
    cid = lax.axis_index("c")
    sid = lax.axis_index("s")
    wid = cid * 16 + sid
    base = wid * _PER_W
    pos_base = lax.rem(base, _S)

    pltpu.sync_copy(x_hbm.at[wid], idx_v)

    for c in range(_NCHUNK):
      pcopy = pltpu.async_copy(
          pos_hbm.at[pl.ds(pos_base + c * _CHUNK, _CHUNK)], acc_v, psem)
      gcopy = pltpu.async_copy(tok_hbm.at[idx_v.at[c]], tok_v, gsem)
      pcopy.wait()
      gcopy.wait()

      def row_body(r, carry):
        for j in range(_VECS):
          sl = pl.ds(j * _LANES, _LANES)
          plsc.addupdate(acc_v.at[r, sl], tok_v[r, sl])
        return carry

      lax.fori_loop(0, _CHUNK, row_body, 0)

      pltpu.sync_copy(acc_v, out_hbm.at[pl.ds(base + c * _CHUNK, _CHUNK)])

  return embed


_EMBED = _build_sc_kernel()


def kernel(x, tok_table, pos_table):
  x32 = x.reshape(-1).astype(jnp.int32).reshape(_NW, _NCHUNK, _CHUNK)
  out = _EMBED(x32, tok_table, pos_table)
  return out.reshape(_B, _S, _D)


# speedup vs baseline: 1.2737x; 1.2737x over previous
import functools

import jax
import jax.numpy as jnp
from jax import lax
from jax.experimental import pallas as pl
from jax.experimental.pallas import tpu as pltpu
from jax.experimental.pallas import tpu_sc as plsc

_VOCAB = 100000
_D = 768
_B = 4
_S = 4096
_N = _B * _S
_NW = 32
_PER_W = _N // _NW
_CHUNK = 32
_NCHUNK = _PER_W // _CHUNK
_LANES = 16
_VECS = _D // _LANES


def _build_sc_kernel():
  mesh = plsc.VectorSubcoreMesh(core_axis_name="c", subcore_axis_name="s")

  @functools.partial(
      pl.kernel,
      mesh=mesh,
      out_type=jax.ShapeDtypeStruct((_N, _D), jnp.float32),
      scratch_types=[
          pltpu.VMEM((_NCHUNK, _CHUNK), jnp.int32),
          pltpu.VMEM((_CHUNK, _D), jnp.float32),
          pltpu.VMEM((_CHUNK, _D), jnp.float32),
          pltpu.VMEM((_CHUNK, _D), jnp.float32),
          pltpu.VMEM((_CHUNK, _D), jnp.float32),
          pltpu.SemaphoreType.DMA,
          pltpu.SemaphoreType.DMA,
          pltpu.SemaphoreType.DMA,
          pltpu.SemaphoreType.DMA,
          pltpu.SemaphoreType.DMA,
          pltpu.SemaphoreType.DMA,
      ],
  )
  def embed(x_hbm, tok_hbm, pos_hbm, out_hbm, idx_v,
            tok0, tok1, acc0, acc1, g0, g1, p0, p1, w0, w1):
    tok_v = [tok0, tok1]
    acc_v = [acc0, acc1]
    gsem = [g0, g1]
    psem = [p0, p1]
    wsem = [w0, w1]

    cid = lax.axis_index("c")
    sid = lax.axis_index("s")
    wid = cid * 16 + sid
    base = wid * _PER_W
    pos_base = lax.rem(base, _S)

    pltpu.sync_copy(x_hbm.at[wid], idx_v)

    def start(c):
      b = c % 2
      ph = pltpu.async_copy(
          pos_hbm.at[pl.ds(pos_base + c * _CHUNK, _CHUNK)], acc_v[b], psem[b])
      gh = pltpu.async_copy(tok_hbm.at[idx_v.at[c]], tok_v[b], gsem[b])
      return ph, gh

    pend = {0: start(0)}
    wr = [None, None]
    for c in range(_NCHUNK):
      b = c % 2
      nb = (c + 1) % 2
      if c + 1 < _NCHUNK:
        if wr[nb] is not None:
          wr[nb].wait()
          wr[nb] = None
        pend[c + 1] = start(c + 1)
      ph, gh = pend.pop(c)
      ph.wait()
      gh.wait()

      def row_body(r, carry, _b=b):
        for j in range(_VECS):
          sl = pl.ds(j * _LANES, _LANES)
          plsc.addupdate(acc_v[_b].at[r, sl], tok_v[_b][r, sl])
        return carry

      lax.fori_loop(0, _CHUNK, row_body, 0)

      wr[b] = pltpu.async_copy(
          acc_v[b], out_hbm.at[pl.ds(base + c * _CHUNK, _CHUNK)], wsem[b])

    for h in wr:
      if h is not None:
        h.wait()

  return embed


_EMBED = _build_sc_kernel()


def kernel(x, tok_table, pos_table):
  x32 = x.reshape(-1).astype(jnp.int32).reshape(_NW, _NCHUNK, _CHUNK)
  out = _EMBED(x32, tok_table, pos_table)
  return out.reshape(_B, _S, _D)


# speedup vs baseline: 1.4212x; 1.1158x over previous
import functools

import jax
import jax.numpy as jnp
from jax import lax
from jax.experimental import pallas as pl
from jax.experimental.pallas import tpu as pltpu
from jax.experimental.pallas import tpu_sc as plsc

_VOCAB = 100000
_D = 768
_B = 4
_S = 4096
_N = _B * _S
_NW = 32
_SEQ_W = _S // _NW
_C = 32
_K = _SEQ_W // _C
_NT = _K * _B
_LANES = 16
_VECS = _D // _LANES


def _build_sc_kernel():
  mesh = plsc.VectorSubcoreMesh(core_axis_name="c", subcore_axis_name="s")

  @functools.partial(
      pl.kernel,
      mesh=mesh,
      out_type=jax.ShapeDtypeStruct((_N, _D), jnp.float32),
      scratch_types=[
          pltpu.VMEM((_NT, _C), jnp.int32),
          pltpu.VMEM((_C, _D), jnp.float32),
          pltpu.VMEM((_C, _D), jnp.float32),
          pltpu.VMEM((_C, _D), jnp.float32),
          pltpu.VMEM((_C, _D), jnp.float32),
          pltpu.VMEM((_C, _D), jnp.float32),
          pltpu.SemaphoreType.DMA,
          pltpu.SemaphoreType.DMA,
          pltpu.SemaphoreType.DMA,
          pltpu.SemaphoreType.DMA,
          pltpu.SemaphoreType.DMA,
          pltpu.SemaphoreType.DMA,
          pltpu.SemaphoreType.DMA,
      ],
  )
  def embed(x_hbm, tok_hbm, pos_hbm, out_hbm, idx_v,
            t0, t1, pb0, pb1, pb2, g0, g1, w0, w1, p0, p1, p2):
    tok = [t0, t1]
    pos_buf = [pb0, pb1, pb2]
    gsem = [g0, g1]
    wsem = [w0, w1]
    psem = [p0, p1, p2]

    cid = lax.axis_index("c")
    sid = lax.axis_index("s")
    wid = cid * 16 + sid
    seq_base = wid * _SEQ_W

    def start_pos(k):
      return pltpu.async_copy(
          pos_hbm.at[pl.ds(seq_base + k * _C, _C)], pos_buf[k % 3],
          psem[k % 3])

    def start_gather(t):
      return pltpu.async_copy(
          tok_hbm.at[idx_v.at[t]], tok[t % 2], gsem[t % 2])

    def start_write(t):
      k, bb = t // _B, t % _B
      return pltpu.async_copy(
          tok[t % 2], out_hbm.at[pl.ds(bb * _S + seq_base + k * _C, _C)],
          wsem[t % 2])

    def compute(t):
      k = t // _B

      @plsc.parallel_loop(0, _C)
      def row_body(r):
        for j in range(_VECS):
          sl = pl.ds(j * _LANES, _LANES)
          plsc.addupdate(tok[t % 2].at[r, sl], pos_buf[k % 3][r, sl])

    pltpu.sync_copy(x_hbm.at[wid], idx_v)
    ph = [start_pos(0), None, None]
    gh = {0: start_gather(0)}
    wh = {}

    for t in range(_NT):
      k = t // _B
      if t % _B == 0 and k + 1 < _K:
        ph[(k + 1) % 3] = start_pos(k + 1)
      if t + 1 < _NT:
        if t - 1 in wh:
          wh.pop(t - 1).wait()
        gh[t + 1] = start_gather(t + 1)
      gh.pop(t).wait()
      if t % _B == 0:
        ph[k % 3].wait()
      compute(t)
      wh[t] = start_write(t)

    for t in sorted(wh):
      wh.pop(t).wait()

  return embed


_EMBED = _build_sc_kernel()


def kernel(x, tok_table, pos_table):
  x32 = (x.astype(jnp.int32)
         .reshape(_B, _NW, _K, _C)
         .transpose(1, 2, 0, 3)
         .reshape(_NW, _NT, _C))
  out = _EMBED(x32, tok_table, pos_table)
  return out.reshape(_B, _S, _D)
